# baseline (speedup 1.0000x reference)
import functools

import jax
import jax.numpy as jnp
from jax import lax
from jax.experimental import pallas as pl
from jax.experimental.pallas import tpu as pltpu
from jax.experimental.pallas import tpu_sc as plsc

N_NODES = 10000
N_EDGES = 320000
D_FEAT = 128
D_HID = 64
D_TGT = 64

NC = 2
NS = 16
NW = NC * NS
E_PER_W = N_EDGES // NW
K = 125
CHUNKS = E_PER_W // K
ROWS_PER_SUB = N_NODES // NS
WB_CHUNK = 125
WB_ITERS = ROWS_PER_SUB // WB_CHUNK


def _head_body(f_ref, w_ref, b_ref, o_ref):
    o_ref[...] = jnp.maximum(
        jnp.dot(f_ref[...], w_ref[...], preferred_element_type=jnp.float32)
        + b_ref[...], 0.0)


def _head(feature, W0, b0):
    return pl.pallas_call(
        _head_body,
        grid=(10,),
        in_specs=[
            pl.BlockSpec((1000, D_FEAT), lambda i: (i, 0)),
            pl.BlockSpec((D_FEAT, D_HID), lambda i: (0, 0)),
            pl.BlockSpec((1, D_HID), lambda i: (0, 0)),
        ],
        out_specs=pl.BlockSpec((1000, D_HID), lambda i: (i, 0)),
        out_shape=jax.ShapeDtypeStruct((N_NODES, D_HID), jnp.float32),
    )(feature, W0, b0.reshape(1, D_HID))


def _seg_body(h0_hbm, src_hbm, dst_hbm, out_hbm, srcv, dstv, rows_a, rows_b,
              zbuf, agg, sem_a, sem_b):
    c = lax.axis_index("c")
    s = lax.axis_index("s")
    wid = s * NC + c
    my_row0 = s * ROWS_PER_SUB

    def zero_row(i, carry):
        for cc in range(D_HID // 16):
            zbuf[i, pl.ds(cc * 16, 16)] = jnp.zeros((16,), jnp.float32)
        return carry
    lax.fori_loop(0, WB_CHUNK, zero_row, 0)

    def zero_slab(p, carry):
        pltpu.sync_copy(zbuf, agg.at[pl.ds(my_row0 + p * WB_CHUNK, WB_CHUNK)])
        return carry
    lax.fori_loop(0, WB_ITERS, zero_slab, 0)

    pltpu.sync_copy(src_hbm.at[wid], srcv)
    pltpu.sync_copy(dst_hbm.at[wid], dstv)

    plsc.subcore_barrier()

    pltpu.async_copy(h0_hbm.at[srcv.at[0]], rows_a, sem_a)

    def edge_pair(i, carry):
        pltpu.async_copy(h0_hbm.at[srcv.at[2 * i + 1]], rows_b, sem_b)
        pltpu.make_async_copy(h0_hbm.at[srcv.at[2 * i]], rows_a, sem_a).wait()
        pltpu.sync_copy(rows_a, agg.at[dstv.at[2 * i]], add=True)

        @pl.when(i < CHUNKS // 2 - 1)
        def _():
            pltpu.async_copy(h0_hbm.at[srcv.at[2 * i + 2]], rows_a, sem_a)
        pltpu.make_async_copy(h0_hbm.at[srcv.at[2 * i + 1]], rows_b, sem_b).wait()
        pltpu.sync_copy(rows_b, agg.at[dstv.at[2 * i + 1]], add=True)
        return carry
    lax.fori_loop(0, CHUNKS // 2, edge_pair, 0)

    plsc.subcore_barrier()

    def writeback(p, carry):
        r0 = my_row0 + p * WB_CHUNK
        pltpu.sync_copy(agg.at[pl.ds(r0, WB_CHUNK)], zbuf)
        pltpu.sync_copy(zbuf, out_hbm.at[c, pl.ds(r0, WB_CHUNK)])
        return carry
    lax.fori_loop(0, WB_ITERS, writeback, 0)


_seg_sum = functools.partial(
    pl.kernel,
    out_type=jax.ShapeDtypeStruct((NC, N_NODES, D_HID), jnp.float32),
    mesh=plsc.VectorSubcoreMesh(core_axis_name="c", subcore_axis_name="s"),
    compiler_params=pltpu.CompilerParams(use_tc_tiling_on_sc=False),
    scratch_types=[
        pltpu.VMEM((CHUNKS, K), jnp.int32),
        pltpu.VMEM((CHUNKS, K), jnp.int32),
        pltpu.VMEM((K, D_HID), jnp.float32),
        pltpu.VMEM((K, D_HID), jnp.float32),
        pltpu.VMEM((WB_CHUNK, D_HID), jnp.float32),
        pltpu.VMEM_SHARED((N_NODES, D_HID), jnp.float32),
        pltpu.SemaphoreType.DMA,
        pltpu.SemaphoreType.DMA,
    ],
)(_seg_body)


TB = 1000
NTB = N_NODES // TB


def _tail_body(h0_ref, a0_ref, a1_ref, w1_ref, b1_ref, w2_ref, b2_ref,
               w3_ref, b3_ref, g_ref, be_ref, o_ref, s1_ref, s2_ref):
    i = pl.program_id(0)

    @pl.when(i < NTB)
    def _():
        x = h0_ref[...] + a0_ref[...] + a1_ref[...]
        h1 = jnp.maximum(
            jnp.dot(x, w1_ref[...], preferred_element_type=jnp.float32)
            + b1_ref[...], 0.0)
        h2 = (jnp.dot(h1, w2_ref[...], preferred_element_type=jnp.float32)
              + b2_ref[...])
        t = (jnp.dot(h2, w3_ref[...], preferred_element_type=jnp.float32)
             + b3_ref[...])
        o_ref[pl.ds(i * TB, TB), :] = t
        ps = jnp.sum(t, axis=0, keepdims=True)
        pss = jnp.sum(t * t, axis=0, keepdims=True)

        @pl.when(i == 0)
        def _():
            s1_ref[...] = jnp.zeros_like(s1_ref)
            s2_ref[...] = jnp.zeros_like(s2_ref)
        s1_ref[...] += ps
        s2_ref[...] += pss

    @pl.when(i == NTB)
    def _():
        mean = s1_ref[...] * (1.0 / N_NODES)
        var = s2_ref[...] * (1.0 / N_NODES) - mean * mean
        scale = g_ref[...] * lax.rsqrt(var + 1e-5)
        shift = be_ref[...] - mean * scale
        o_ref[...] = o_ref[...] * scale + shift


def _tail(h0, a0, a1, W1, b1, W2, b2, W3, b3, gamma, beta):
    blk = lambda i: (jnp.minimum(i, NTB - 1), 0)
    full = lambda i: (0, 0)
    return pl.pallas_call(
        _tail_body,
        grid=(NTB + 1,),
        in_specs=[
            pl.BlockSpec((TB, D_HID), blk),
            pl.BlockSpec((TB, D_HID), blk),
            pl.BlockSpec((TB, D_HID), blk),
            pl.BlockSpec((D_HID, D_HID), full),
            pl.BlockSpec((1, D_HID), full),
            pl.BlockSpec((D_HID, D_HID), full),
            pl.BlockSpec((1, D_HID), full),
            pl.BlockSpec((D_HID, D_TGT), full),
            pl.BlockSpec((1, D_TGT), full),
            pl.BlockSpec((1, D_TGT), full),
            pl.BlockSpec((1, D_TGT), full),
        ],
        out_specs=pl.BlockSpec((N_NODES, D_TGT), full),
        out_shape=jax.ShapeDtypeStruct((N_NODES, D_TGT), jnp.float32),
        scratch_shapes=[
            pltpu.VMEM((1, D_TGT), jnp.float32),
            pltpu.VMEM((1, D_TGT), jnp.float32),
        ],
    )(h0, a0, a1, W1, b1.reshape(1, D_HID), W2, b2.reshape(1, D_HID),
      W3, b3.reshape(1, D_TGT), gamma.reshape(1, D_TGT), beta.reshape(1, D_TGT))


def kernel(feature, edge_index, W0, b0, W1, b1, W2, b2, W3, b3, gamma, beta):
    src = edge_index[0].astype(jnp.int32).reshape(NW, CHUNKS, K)
    dst = edge_index[1].astype(jnp.int32).reshape(NW, CHUNKS, K)
    h0 = _head(feature, W0, b0)
    parts = _seg_sum(h0, src, dst)
    return _tail(h0, parts[0], parts[1], W1, b1, W2, b2, W3, b3, gamma, beta)


# speedup vs baseline: 1.0518x; 1.0518x over previous
import functools

import jax
import jax.numpy as jnp
from jax import lax
from jax.experimental import pallas as pl
from jax.experimental.pallas import tpu as pltpu
from jax.experimental.pallas import tpu_sc as plsc

N_NODES = 10000
N_EDGES = 320000
D_FEAT = 128
D_HID = 64
D_TGT = 64

NC = 2
NS = 16
NW = NC * NS
E_PER_W = N_EDGES // NW
K = 80
CHUNKS = E_PER_W // K
ROWS_PER_SUB = N_NODES // NS
WB_CHUNK = 125
WB_ITERS = ROWS_PER_SUB // WB_CHUNK


def _head_body(f_ref, w_ref, b_ref, o_ref):
    o_ref[...] = jnp.maximum(
        jnp.dot(f_ref[...], w_ref[...], preferred_element_type=jnp.float32)
        + b_ref[...], 0.0)


def _head(feature, W0, b0):
    return pl.pallas_call(
        _head_body,
        grid=(10,),
        in_specs=[
            pl.BlockSpec((1000, D_FEAT), lambda i: (i, 0)),
            pl.BlockSpec((D_FEAT, D_HID), lambda i: (0, 0)),
            pl.BlockSpec((1, D_HID), lambda i: (0, 0)),
        ],
        out_specs=pl.BlockSpec((1000, D_HID), lambda i: (i, 0)),
        out_shape=jax.ShapeDtypeStruct((N_NODES, D_HID), jnp.float32),
    )(feature, W0, b0.reshape(1, D_HID))


def _seg_body(h0_hbm, ei_hbm, out_hbm, srcv, dstv, rows_a, rows_b,
              zbuf, agg, sem_a, sem_b):
    c = lax.axis_index("c")
    s = lax.axis_index("s")
    wid = s * NC + c
    my_e0 = wid * E_PER_W
    my_row0 = s * ROWS_PER_SUB

    def zero_row(i, carry):
        for cc in range(D_HID // 16):
            zbuf[i, pl.ds(cc * 16, 16)] = jnp.zeros((16,), jnp.float32)
        return carry
    lax.fori_loop(0, WB_CHUNK, zero_row, 0)

    def zero_slab(p, carry):
        pltpu.sync_copy(zbuf, agg.at[pl.ds(my_row0 + p * WB_CHUNK, WB_CHUNK)])
        return carry
    lax.fori_loop(0, WB_ITERS, zero_slab, 0)

    pltpu.sync_copy(ei_hbm.at[0, pl.ds(my_e0, E_PER_W)], srcv)
    pltpu.sync_copy(ei_hbm.at[1, pl.ds(my_e0, E_PER_W)], dstv)

    plsc.subcore_barrier()

    def sidx(j):
        return srcv.at[pl.ds(j * K, K)]

    def didx(j):
        return dstv.at[pl.ds(j * K, K)]

    pltpu.async_copy(h0_hbm.at[sidx(0)], rows_a, sem_a)

    def edge_pair(i, carry):
        pltpu.async_copy(h0_hbm.at[sidx(2 * i + 1)], rows_b, sem_b)
        pltpu.make_async_copy(h0_hbm.at[sidx(2 * i)], rows_a, sem_a).wait()
        pltpu.sync_copy(rows_a, agg.at[didx(2 * i)], add=True)

        @pl.when(i < CHUNKS // 2 - 1)
        def _():
            pltpu.async_copy(h0_hbm.at[sidx(2 * i + 2)], rows_a, sem_a)
        pltpu.make_async_copy(h0_hbm.at[sidx(2 * i + 1)], rows_b, sem_b).wait()
        pltpu.sync_copy(rows_b, agg.at[didx(2 * i + 1)], add=True)
        return carry
    lax.fori_loop(0, CHUNKS // 2, edge_pair, 0)

    pltpu.async_copy(h0_hbm.at[sidx(CHUNKS - 1)], rows_a, sem_a)
    pltpu.make_async_copy(h0_hbm.at[sidx(CHUNKS - 1)], rows_a, sem_a).wait()
    pltpu.sync_copy(rows_a, agg.at[didx(CHUNKS - 1)], add=True)

    plsc.subcore_barrier()

    def writeback(p, carry):
        r0 = my_row0 + p * WB_CHUNK
        pltpu.sync_copy(agg.at[pl.ds(r0, WB_CHUNK)], zbuf)
        pltpu.sync_copy(zbuf, out_hbm.at[c, pl.ds(r0, WB_CHUNK)])
        return carry
    lax.fori_loop(0, WB_ITERS, writeback, 0)


_seg_sum = functools.partial(
    pl.kernel,
    out_type=jax.ShapeDtypeStruct((NC, N_NODES, D_HID), jnp.float32),
    mesh=plsc.VectorSubcoreMesh(core_axis_name="c", subcore_axis_name="s"),
    compiler_params=pltpu.CompilerParams(use_tc_tiling_on_sc=False),
    scratch_types=[
        pltpu.VMEM((E_PER_W,), jnp.int32),
        pltpu.VMEM((E_PER_W,), jnp.int32),
        pltpu.VMEM((K, D_HID), jnp.float32),
        pltpu.VMEM((K, D_HID), jnp.float32),
        pltpu.VMEM((WB_CHUNK, D_HID), jnp.float32),
        pltpu.VMEM_SHARED((N_NODES, D_HID), jnp.float32),
        pltpu.SemaphoreType.DMA,
        pltpu.SemaphoreType.DMA,
    ],
)(_seg_body)


TB = 1000
NTB = N_NODES // TB


def _tail_body(h0_ref, parts_ref, w1_ref, b1_ref, w2_ref, b2_ref,
               w3_ref, b3_ref, g_ref, be_ref, o_ref, s1_ref, s2_ref):
    i = pl.program_id(0)

    @pl.when(i < NTB)
    def _():
        x = h0_ref[...] + parts_ref[0] + parts_ref[1]
        h1 = jnp.maximum(
            jnp.dot(x, w1_ref[...], preferred_element_type=jnp.float32)
            + b1_ref[...], 0.0)
        h2 = (jnp.dot(h1, w2_ref[...], preferred_element_type=jnp.float32)
              + b2_ref[...])
        t = (jnp.dot(h2, w3_ref[...], preferred_element_type=jnp.float32)
             + b3_ref[...])
        o_ref[pl.ds(i * TB, TB), :] = t
        ps = jnp.sum(t, axis=0, keepdims=True)
        pss = jnp.sum(t * t, axis=0, keepdims=True)

        @pl.when(i == 0)
        def _():
            s1_ref[...] = jnp.zeros_like(s1_ref)
            s2_ref[...] = jnp.zeros_like(s2_ref)
        s1_ref[...] += ps
        s2_ref[...] += pss

    @pl.when(i == NTB)
    def _():
        mean = s1_ref[...] * (1.0 / N_NODES)
        var = s2_ref[...] * (1.0 / N_NODES) - mean * mean
        scale = g_ref[...] * lax.rsqrt(var + 1e-5)
        shift = be_ref[...] - mean * scale
        o_ref[...] = o_ref[...] * scale + shift


def _tail(h0, parts, W1, b1, W2, b2, W3, b3, gamma, beta):
    blk = lambda i: (jnp.minimum(i, NTB - 1), 0)
    blk3 = lambda i: (0, jnp.minimum(i, NTB - 1), 0)
    full = lambda i: (0, 0)
    return pl.pallas_call(
        _tail_body,
        grid=(NTB + 1,),
        in_specs=[
            pl.BlockSpec((TB, D_HID), blk),
            pl.BlockSpec((NC, TB, D_HID), blk3),
            pl.BlockSpec((D_HID, D_HID), full),
            pl.BlockSpec((1, D_HID), full),
            pl.BlockSpec((D_HID, D_HID), full),
            pl.BlockSpec((1, D_HID), full),
            pl.BlockSpec((D_HID, D_TGT), full),
            pl.BlockSpec((1, D_TGT), full),
            pl.BlockSpec((1, D_TGT), full),
            pl.BlockSpec((1, D_TGT), full),
        ],
        out_specs=pl.BlockSpec((N_NODES, D_TGT), full),
        out_shape=jax.ShapeDtypeStruct((N_NODES, D_TGT), jnp.float32),
        scratch_shapes=[
            pltpu.VMEM((1, D_TGT), jnp.float32),
            pltpu.VMEM((1, D_TGT), jnp.float32),
        ],
    )(h0, parts, W1, b1.reshape(1, D_HID), W2, b2.reshape(1, D_HID),
      W3, b3.reshape(1, D_TGT), gamma.reshape(1, D_TGT), beta.reshape(1, D_TGT))


def kernel(feature, edge_index, W0, b0, W1, b1, W2, b2, W3, b3, gamma, beta):
    ei = edge_index.astype(jnp.int32)
    h0 = _head(feature, W0, b0)
    parts = _seg_sum(h0, ei)
    return _tail(h0, parts, W1, b1, W2, b2, W3, b3, gamma, beta)


# speedup vs baseline: 1.1556x; 1.0987x over previous
import functools

import jax
import jax.numpy as jnp
from jax import lax
from jax.experimental import pallas as pl
from jax.experimental.pallas import tpu as pltpu
from jax.experimental.pallas import tpu_sc as plsc

N_NODES = 10000
N_EDGES = 320000
D_FEAT = 128
D_HID = 64
D_TGT = 64

NC = 2
NS = 16
NW = NC * NS
K = 128
CHUNKS = 78
N_EXTRA = (N_EDGES - NW * CHUNKS * K) // K
SLAB = CHUNKS * K
ROWS_PER_SUB = N_NODES // NS
WB_CHUNK = 125
WB_ITERS = ROWS_PER_SUB // WB_CHUNK


def _head_body(f_ref, w_ref, b_ref, o_ref):
    o_ref[...] = jnp.maximum(
        jnp.dot(f_ref[...], w_ref[...], preferred_element_type=jnp.float32)
        + b_ref[...], 0.0)


def _head(feature, W0, b0):
    return pl.pallas_call(
        _head_body,
        grid=(10,),
        in_specs=[
            pl.BlockSpec((1000, D_FEAT), lambda i: (i, 0)),
            pl.BlockSpec((D_FEAT, D_HID), lambda i: (0, 0)),
            pl.BlockSpec((1, D_HID), lambda i: (0, 0)),
        ],
        out_specs=pl.BlockSpec((1000, D_HID), lambda i: (i, 0)),
        out_shape=jax.ShapeDtypeStruct((N_NODES, D_HID), jnp.float32),
    )(feature, W0, b0.reshape(1, D_HID))


def _seg_body(h0_hbm, ei_hbm, out_hbm, srcv, dstv, srcx, dstx, rows_a, rows_b,
              zbuf, agg, sem_a, sem_b):
    c = lax.axis_index("c")
    s = lax.axis_index("s")
    wid = s * NC + c
    my_e0 = wid * SLAB
    my_row0 = s * ROWS_PER_SUB

    def zero_row(i, carry):
        for cc in range(D_HID // 16):
            zbuf[i, pl.ds(cc * 16, 16)] = jnp.zeros((16,), jnp.float32)
        return carry
    lax.fori_loop(0, WB_CHUNK, zero_row, 0)

    def zero_slab(p, carry):
        pltpu.sync_copy(zbuf, agg.at[pl.ds(my_row0 + p * WB_CHUNK, WB_CHUNK)])
        return carry
    lax.fori_loop(0, WB_ITERS, zero_slab, 0)

    pltpu.sync_copy(ei_hbm.at[0, pl.ds(my_e0, SLAB)], srcv)
    pltpu.sync_copy(ei_hbm.at[1, pl.ds(my_e0, SLAB)], dstv)
    extra0 = NW * CHUNKS * K + wid * K

    @pl.when(wid < N_EXTRA)
    def _():
        pltpu.sync_copy(ei_hbm.at[0, pl.ds(extra0, K)], srcx)
        pltpu.sync_copy(ei_hbm.at[1, pl.ds(extra0, K)], dstx)

    plsc.subcore_barrier()

    def sidx(j):
        return srcv.at[pl.ds(j * K, K)]

    def didx(j):
        return dstv.at[pl.ds(j * K, K)]

    pltpu.async_copy(h0_hbm.at[sidx(0)], rows_a, sem_a)

    def edge_pair(i, carry):
        pltpu.async_copy(h0_hbm.at[sidx(2 * i + 1)], rows_b, sem_b)
        pltpu.make_async_copy(h0_hbm.at[sidx(2 * i)], rows_a, sem_a).wait()
        pltpu.sync_copy(rows_a, agg.at[didx(2 * i)], add=True)

        @pl.when(i < CHUNKS // 2 - 1)
        def _():
            pltpu.async_copy(h0_hbm.at[sidx(2 * i + 2)], rows_a, sem_a)
        pltpu.make_async_copy(h0_hbm.at[sidx(2 * i + 1)], rows_b, sem_b).wait()
        pltpu.sync_copy(rows_b, agg.at[didx(2 * i + 1)], add=True)
        return carry
    lax.fori_loop(0, CHUNKS // 2, edge_pair, 0)

    @pl.when(wid < N_EXTRA)
    def _():
        pltpu.async_copy(h0_hbm.at[srcx], rows_a, sem_a)
        pltpu.make_async_copy(h0_hbm.at[srcx], rows_a, sem_a).wait()
        pltpu.sync_copy(rows_a, agg.at[dstx], add=True)

    plsc.subcore_barrier()

    def writeback(p, carry):
        r0 = my_row0 + p * WB_CHUNK
        pltpu.sync_copy(agg.at[pl.ds(r0, WB_CHUNK)], zbuf)
        pltpu.sync_copy(zbuf, out_hbm.at[c, pl.ds(r0, WB_CHUNK)])
        return carry
    lax.fori_loop(0, WB_ITERS, writeback, 0)


_seg_sum = functools.partial(
    pl.kernel,
    out_type=jax.ShapeDtypeStruct((NC, N_NODES, D_HID), jnp.float32),
    mesh=plsc.VectorSubcoreMesh(core_axis_name="c", subcore_axis_name="s"),
    compiler_params=pltpu.CompilerParams(use_tc_tiling_on_sc=False),
    scratch_types=[
        pltpu.VMEM((SLAB,), jnp.int32),
        pltpu.VMEM((SLAB,), jnp.int32),
        pltpu.VMEM((K,), jnp.int32),
        pltpu.VMEM((K,), jnp.int32),
        pltpu.VMEM((K, D_HID), jnp.float32),
        pltpu.VMEM((K, D_HID), jnp.float32),
        pltpu.VMEM((WB_CHUNK, D_HID), jnp.float32),
        pltpu.VMEM_SHARED((N_NODES, D_HID), jnp.float32),
        pltpu.SemaphoreType.DMA,
        pltpu.SemaphoreType.DMA,
    ],
)(_seg_body)


TB = 1000
NTB = N_NODES // TB


def _tail_body(h0_ref, parts_ref, w1_ref, b1_ref, w2_ref, b2_ref,
               w3_ref, b3_ref, g_ref, be_ref, o_ref, s1_ref, s2_ref):
    i = pl.program_id(0)

    @pl.when(i < NTB)
    def _():
        x = h0_ref[...] + parts_ref[0] + parts_ref[1]
        h1 = jnp.maximum(
            jnp.dot(x, w1_ref[...], preferred_element_type=jnp.float32)
            + b1_ref[...], 0.0)
        h2 = (jnp.dot(h1, w2_ref[...], preferred_element_type=jnp.float32)
              + b2_ref[...])
        t = (jnp.dot(h2, w3_ref[...], preferred_element_type=jnp.float32)
             + b3_ref[...])
        o_ref[pl.ds(i * TB, TB), :] = t
        ps = jnp.sum(t, axis=0, keepdims=True)
        pss = jnp.sum(t * t, axis=0, keepdims=True)

        @pl.when(i == 0)
        def _():
            s1_ref[...] = jnp.zeros_like(s1_ref)
            s2_ref[...] = jnp.zeros_like(s2_ref)
        s1_ref[...] += ps
        s2_ref[...] += pss

    @pl.when(i == NTB)
    def _():
        mean = s1_ref[...] * (1.0 / N_NODES)
        var = s2_ref[...] * (1.0 / N_NODES) - mean * mean
        scale = g_ref[...] * lax.rsqrt(var + 1e-5)
        shift = be_ref[...] - mean * scale
        o_ref[...] = o_ref[...] * scale + shift


def _tail(h0, parts, W1, b1, W2, b2, W3, b3, gamma, beta):
    blk = lambda i: (jnp.minimum(i, NTB - 1), 0)
    blk3 = lambda i: (0, jnp.minimum(i, NTB - 1), 0)
    full = lambda i: (0, 0)
    return pl.pallas_call(
        _tail_body,
        grid=(NTB + 1,),
        in_specs=[
            pl.BlockSpec((TB, D_HID), blk),
            pl.BlockSpec((NC, TB, D_HID), blk3),
            pl.BlockSpec((D_HID, D_HID), full),
            pl.BlockSpec((1, D_HID), full),
            pl.BlockSpec((D_HID, D_HID), full),
            pl.BlockSpec((1, D_HID), full),
            pl.BlockSpec((D_HID, D_TGT), full),
            pl.BlockSpec((1, D_TGT), full),
            pl.BlockSpec((1, D_TGT), full),
            pl.BlockSpec((1, D_TGT), full),
        ],
        out_specs=pl.BlockSpec((N_NODES, D_TGT), full),
        out_shape=jax.ShapeDtypeStruct((N_NODES, D_TGT), jnp.float32),
        scratch_shapes=[
            pltpu.VMEM((1, D_TGT), jnp.float32),
            pltpu.VMEM((1, D_TGT), jnp.float32),
        ],
    )(h0, parts, W1, b1.reshape(1, D_HID), W2, b2.reshape(1, D_HID),
      W3, b3.reshape(1, D_TGT), gamma.reshape(1, D_TGT), beta.reshape(1, D_TGT))


def kernel(feature, edge_index, W0, b0, W1, b1, W2, b2, W3, b3, gamma, beta):
    ei = edge_index.astype(jnp.int32)
    h0 = _head(feature, W0, b0)
    parts = _seg_sum(h0, ei)
    return _tail(h0, parts, W1, b1, W2, b2, W3, b3, gamma, beta)


# speedup vs baseline: 1.2221x; 1.0576x over previous
import functools

import jax
import jax.numpy as jnp
from jax import lax
from jax.experimental import pallas as pl
from jax.experimental.pallas import tpu as pltpu
from jax.experimental.pallas import tpu_sc as plsc

N_NODES = 10000
N_EDGES = 320000
D_FEAT = 128
D_HID = 64
D_TGT = 64

NC = 2
NS = 16
NW = NC * NS
K = 128
CHUNKS = 78
N_EXTRA = (N_EDGES - NW * CHUNKS * K) // K
SLAB = CHUNKS * K
ROWS_PER_SUB = N_NODES // NS
WB_CHUNK = 125
WB_ITERS = ROWS_PER_SUB // WB_CHUNK


def _head_body(f_ref, w_ref, b_ref, o_ref):
    o_ref[...] = jnp.maximum(
        jnp.dot(f_ref[...], w_ref[...], preferred_element_type=jnp.float32)
        + b_ref[...], 0.0)


D_PAD = 128


def _head(feature, W0, b0):
    return pl.pallas_call(
        _head_body,
        grid=(10,),
        in_specs=[
            pl.BlockSpec((1000, D_FEAT), lambda i: (i, 0)),
            pl.BlockSpec((D_FEAT, D_HID), lambda i: (0, 0)),
            pl.BlockSpec((1, D_HID), lambda i: (0, 0)),
        ],
        out_specs=pl.BlockSpec((1000, D_HID), lambda i: (i, 0)),
        out_shape=jax.ShapeDtypeStruct((N_NODES, D_HID), jnp.float32),
    )(feature, W0, b0.reshape(1, D_HID))


def _seg_body(h0_hbm, ei_hbm, out_hbm, srcv, dstv, srcx, dstx, rows_a, rows_b,
              zbuf, agg, sem_a, sem_b):
    c = lax.axis_index("c")
    s = lax.axis_index("s")
    wid = s * NC + c
    my_e0 = wid * SLAB
    my_row0 = s * ROWS_PER_SUB

    def zero_row(i, carry):
        for cc in range(D_PAD // 16):
            zbuf[i, pl.ds(cc * 16, 16)] = jnp.zeros((16,), jnp.float32)
        return carry
    lax.fori_loop(0, WB_CHUNK, zero_row, 0)

    zvalid = zbuf.at[:, pl.ds(0, D_HID)]

    def zero_slab(p, carry):
        pltpu.sync_copy(zvalid, agg.at[pl.ds(my_row0 + p * WB_CHUNK, WB_CHUNK)])
        return carry
    lax.fori_loop(0, WB_ITERS, zero_slab, 0)

    pltpu.sync_copy(ei_hbm.at[0, pl.ds(my_e0, SLAB)], srcv)
    pltpu.sync_copy(ei_hbm.at[1, pl.ds(my_e0, SLAB)], dstv)
    extra0 = NW * CHUNKS * K + wid * K

    @pl.when(wid < N_EXTRA)
    def _():
        pltpu.sync_copy(ei_hbm.at[0, pl.ds(extra0, K)], srcx)
        pltpu.sync_copy(ei_hbm.at[1, pl.ds(extra0, K)], dstx)

    plsc.subcore_barrier()

    def sidx(j):
        return srcv.at[pl.ds(j * K, K)]

    def didx(j):
        return dstv.at[pl.ds(j * K, K)]

    pltpu.async_copy(h0_hbm.at[sidx(0)], rows_a, sem_a)

    def edge_pair(i, carry):
        pltpu.async_copy(h0_hbm.at[sidx(2 * i + 1)], rows_b, sem_b)
        pltpu.make_async_copy(h0_hbm.at[sidx(2 * i)], rows_a, sem_a).wait()
        pltpu.sync_copy(rows_a, agg.at[didx(2 * i)], add=True)

        @pl.when(i < CHUNKS // 2 - 1)
        def _():
            pltpu.async_copy(h0_hbm.at[sidx(2 * i + 2)], rows_a, sem_a)
        pltpu.make_async_copy(h0_hbm.at[sidx(2 * i + 1)], rows_b, sem_b).wait()
        pltpu.sync_copy(rows_b, agg.at[didx(2 * i + 1)], add=True)
        return carry
    lax.fori_loop(0, CHUNKS // 2, edge_pair, 0)

    @pl.when(wid < N_EXTRA)
    def _():
        pltpu.async_copy(h0_hbm.at[srcx], rows_a, sem_a)
        pltpu.make_async_copy(h0_hbm.at[srcx], rows_a, sem_a).wait()
        pltpu.sync_copy(rows_a, agg.at[dstx], add=True)

    plsc.subcore_barrier()

    def writeback(p, carry):
        r0 = my_row0 + p * WB_CHUNK
        pltpu.sync_copy(agg.at[pl.ds(r0, WB_CHUNK)], zvalid)
        pltpu.sync_copy(zbuf, out_hbm.at[c, pl.ds(r0, WB_CHUNK)])
        return carry
    lax.fori_loop(0, WB_ITERS, writeback, 0)


_seg_sum = functools.partial(
    pl.kernel,
    out_type=jax.ShapeDtypeStruct((NC, N_NODES, D_PAD), jnp.float32),
    mesh=plsc.VectorSubcoreMesh(core_axis_name="c", subcore_axis_name="s"),
    compiler_params=pltpu.CompilerParams(use_tc_tiling_on_sc=False),
    scratch_types=[
        pltpu.VMEM((SLAB,), jnp.int32),
        pltpu.VMEM((SLAB,), jnp.int32),
        pltpu.VMEM((K,), jnp.int32),
        pltpu.VMEM((K,), jnp.int32),
        pltpu.VMEM((K, D_HID), jnp.float32),
        pltpu.VMEM((K, D_HID), jnp.float32),
        pltpu.VMEM((WB_CHUNK, D_PAD), jnp.float32),
        pltpu.VMEM_SHARED((N_NODES, D_HID), jnp.float32),
        pltpu.SemaphoreType.DMA,
        pltpu.SemaphoreType.DMA,
    ],
)(_seg_body)


TB = 1000
NTB = N_NODES // TB


def _tail_body(h0_ref, parts_ref, w1_ref, b1_ref, w2_ref, b2_ref,
               w3_ref, b3_ref, g_ref, be_ref, o_ref, s1_ref, s2_ref):
    i = pl.program_id(0)

    @pl.when(i < NTB)
    def _():
        h0b = h0_ref[...]
        x = jnp.pad(h0b, ((0, 0), (0, D_PAD - D_HID))) + parts_ref[0] + parts_ref[1]
        h1 = jnp.maximum(
            jnp.dot(x, w1_ref[...], preferred_element_type=jnp.float32)
            + b1_ref[...], 0.0)
        h2 = (jnp.dot(h1, w2_ref[...], preferred_element_type=jnp.float32)
              + b2_ref[...])
        t = (jnp.dot(h2, w3_ref[...], preferred_element_type=jnp.float32)
             + b3_ref[...])
        o_ref[pl.ds(i * TB, TB), :] = t
        ps = jnp.sum(t, axis=0, keepdims=True)
        pss = jnp.sum(t * t, axis=0, keepdims=True)

        @pl.when(i == 0)
        def _():
            s1_ref[...] = jnp.zeros_like(s1_ref)
            s2_ref[...] = jnp.zeros_like(s2_ref)
        s1_ref[...] += ps
        s2_ref[...] += pss

    @pl.when(i == NTB)
    def _():
        mean = s1_ref[...] * (1.0 / N_NODES)
        var = s2_ref[...] * (1.0 / N_NODES) - mean * mean
        scale = g_ref[...] * lax.rsqrt(var + 1e-5)
        shift = be_ref[...] - mean * scale
        o_ref[...] = o_ref[...] * scale + shift


def _tail(h0, parts, W1, b1, W2, b2, W3, b3, gamma, beta):
    W1p = jnp.pad(W1, ((0, D_PAD - D_HID), (0, 0)))
    blk = lambda i: (jnp.minimum(i, NTB - 1), 0)
    blk3 = lambda i: (0, jnp.minimum(i, NTB - 1), 0)
    full = lambda i: (0, 0)
    return pl.pallas_call(
        _tail_body,
        grid=(NTB + 1,),
        in_specs=[
            pl.BlockSpec((TB, D_HID), blk),
            pl.BlockSpec((NC, TB, D_PAD), blk3),
            pl.BlockSpec((D_PAD, D_HID), full),
            pl.BlockSpec((1, D_HID), full),
            pl.BlockSpec((D_HID, D_HID), full),
            pl.BlockSpec((1, D_HID), full),
            pl.BlockSpec((D_HID, D_TGT), full),
            pl.BlockSpec((1, D_TGT), full),
            pl.BlockSpec((1, D_TGT), full),
            pl.BlockSpec((1, D_TGT), full),
        ],
        out_specs=pl.BlockSpec((N_NODES, D_TGT), full),
        out_shape=jax.ShapeDtypeStruct((N_NODES, D_TGT), jnp.float32),
        scratch_shapes=[
            pltpu.VMEM((1, D_TGT), jnp.float32),
            pltpu.VMEM((1, D_TGT), jnp.float32),
        ],
    )(h0, parts, W1p, b1.reshape(1, D_HID), W2, b2.reshape(1, D_HID),
      W3, b3.reshape(1, D_TGT), gamma.reshape(1, D_TGT), beta.reshape(1, D_TGT))


def kernel(feature, edge_index, W0, b0, W1, b1, W2, b2, W3, b3, gamma, beta):
    ei = edge_index.astype(jnp.int32)
    h0 = _head(feature, W0, b0)
    parts = _seg_sum(h0, ei)
    return _tail(h0, parts, W1, b1, W2, b2, W3, b3, gamma, beta)


# speedup vs baseline: 1.2854x; 1.0518x over previous
import functools

import jax
import jax.numpy as jnp
from jax import lax
from jax.experimental import pallas as pl
from jax.experimental.pallas import tpu as pltpu
from jax.experimental.pallas import tpu_sc as plsc

N_NODES = 10000
N_EDGES = 320000
D_FEAT = 128
D_HID = 64
D_TGT = 64

NC = 2
NS = 16
NW = NC * NS
K = 128
CHUNKS = 78
N_EXTRA = (N_EDGES - NW * CHUNKS * K) // K
SLAB = CHUNKS * K
ROWS_PER_SUB = N_NODES // NS
WB_CHUNK = 125
WB_ITERS = ROWS_PER_SUB // WB_CHUNK


def _head_body(f_ref, w_ref, b_ref, o_ref):
    o_ref[...] = jnp.maximum(
        jnp.dot(f_ref[...], w_ref[...], preferred_element_type=jnp.float32)
        + b_ref[...], 0.0)


D_PAD = 128


def _head(feature, W0, b0):
    return pl.pallas_call(
        _head_body,
        grid=(5,),
        in_specs=[
            pl.BlockSpec((2000, D_FEAT), lambda i: (i, 0)),
            pl.BlockSpec((D_FEAT, D_HID), lambda i: (0, 0)),
            pl.BlockSpec((1, D_HID), lambda i: (0, 0)),
        ],
        out_specs=pl.BlockSpec((2000, D_HID), lambda i: (i, 0)),
        out_shape=jax.ShapeDtypeStruct((N_NODES, D_HID), jnp.float32),
    )(feature, W0, b0.reshape(1, D_HID))


def _seg_body(h0_hbm, ei_hbm, out_hbm, srcv, dstv, srcx, dstx, rows_a, rows_b,
              zbuf, agg, sem_a, sem_b):
    c = lax.axis_index("c")
    s = lax.axis_index("s")
    wid = s * NC + c
    my_e0 = wid * SLAB
    my_row0 = s * ROWS_PER_SUB

    def zero_row(i, carry):
        for cc in range(D_PAD // 16):
            zbuf[i, pl.ds(cc * 16, 16)] = jnp.zeros((16,), jnp.float32)
        return carry
    lax.fori_loop(0, WB_CHUNK, zero_row, 0)

    zvalid = zbuf.at[:, pl.ds(0, D_HID)]

    def zero_slab(p, carry):
        pltpu.sync_copy(zvalid, agg.at[pl.ds(my_row0 + p * WB_CHUNK, WB_CHUNK)])
        return carry
    lax.fori_loop(0, WB_ITERS, zero_slab, 0)

    pltpu.sync_copy(ei_hbm.at[0, pl.ds(my_e0, SLAB)], srcv)
    pltpu.sync_copy(ei_hbm.at[1, pl.ds(my_e0, SLAB)], dstv)
    extra0 = NW * CHUNKS * K + wid * K

    @pl.when(wid < N_EXTRA)
    def _():
        pltpu.sync_copy(ei_hbm.at[0, pl.ds(extra0, K)], srcx)
        pltpu.sync_copy(ei_hbm.at[1, pl.ds(extra0, K)], dstx)

    plsc.subcore_barrier()

    def sidx(j):
        return srcv.at[pl.ds(j * K, K)]

    def didx(j):
        return dstv.at[pl.ds(j * K, K)]

    pltpu.async_copy(h0_hbm.at[sidx(0)], rows_a, sem_a)

    def edge_pair(i, carry):
        pltpu.async_copy(h0_hbm.at[sidx(2 * i + 1)], rows_b, sem_b)
        pltpu.make_async_copy(h0_hbm.at[sidx(2 * i)], rows_a, sem_a).wait()
        pltpu.sync_copy(rows_a, agg.at[didx(2 * i)], add=True)

        @pl.when(i < CHUNKS // 2 - 1)
        def _():
            pltpu.async_copy(h0_hbm.at[sidx(2 * i + 2)], rows_a, sem_a)
        pltpu.make_async_copy(h0_hbm.at[sidx(2 * i + 1)], rows_b, sem_b).wait()
        pltpu.sync_copy(rows_b, agg.at[didx(2 * i + 1)], add=True)
        return carry
    lax.fori_loop(0, CHUNKS // 2, edge_pair, 0)

    @pl.when(wid < N_EXTRA)
    def _():
        pltpu.async_copy(h0_hbm.at[srcx], rows_a, sem_a)
        pltpu.make_async_copy(h0_hbm.at[srcx], rows_a, sem_a).wait()
        pltpu.sync_copy(rows_a, agg.at[dstx], add=True)

    plsc.subcore_barrier()

    def writeback(p, carry):
        r0 = my_row0 + p * WB_CHUNK
        pltpu.sync_copy(agg.at[pl.ds(r0, WB_CHUNK)], zvalid)
        pltpu.sync_copy(zbuf, out_hbm.at[c, pl.ds(r0, WB_CHUNK)])
        return carry
    lax.fori_loop(0, WB_ITERS, writeback, 0)


_seg_sum = functools.partial(
    pl.kernel,
    out_type=jax.ShapeDtypeStruct((NC, N_NODES, D_PAD), jnp.float32),
    mesh=plsc.VectorSubcoreMesh(core_axis_name="c", subcore_axis_name="s"),
    compiler_params=pltpu.CompilerParams(use_tc_tiling_on_sc=False),
    scratch_types=[
        pltpu.VMEM((SLAB,), jnp.int32),
        pltpu.VMEM((SLAB,), jnp.int32),
        pltpu.VMEM((K,), jnp.int32),
        pltpu.VMEM((K,), jnp.int32),
        pltpu.VMEM((K, D_HID), jnp.float32),
        pltpu.VMEM((K, D_HID), jnp.float32),
        pltpu.VMEM((WB_CHUNK, D_PAD), jnp.float32),
        pltpu.VMEM_SHARED((N_NODES, D_HID), jnp.float32),
        pltpu.SemaphoreType.DMA,
        pltpu.SemaphoreType.DMA,
    ],
)(_seg_body)


TB = 2000
NTB = N_NODES // TB


def _tail_body(h0_ref, parts_ref, w1_ref, b1_ref, w2_ref, b2_ref,
               w3_ref, b3_ref, g_ref, be_ref, o_ref, s1_ref, s2_ref):
    i = pl.program_id(0)

    @pl.when(i < NTB)
    def _():
        h0b = h0_ref[...]
        x = jnp.pad(h0b, ((0, 0), (0, D_PAD - D_HID))) + parts_ref[0] + parts_ref[1]
        h1 = jnp.maximum(
            jnp.dot(x, w1_ref[...], preferred_element_type=jnp.float32)
            + b1_ref[...], 0.0)
        h2 = (jnp.dot(h1, w2_ref[...], preferred_element_type=jnp.float32)
              + b2_ref[...])
        t = (jnp.dot(h2, w3_ref[...], preferred_element_type=jnp.float32)
             + b3_ref[...])
        o_ref[pl.ds(i * TB, TB), :] = t
        ps = jnp.sum(t, axis=0, keepdims=True)
        pss = jnp.sum(t * t, axis=0, keepdims=True)

        @pl.when(i == 0)
        def _():
            s1_ref[...] = jnp.zeros_like(s1_ref)
            s2_ref[...] = jnp.zeros_like(s2_ref)
        s1_ref[...] += ps
        s2_ref[...] += pss

    @pl.when(i == NTB)
    def _():
        mean = s1_ref[...] * (1.0 / N_NODES)
        var = s2_ref[...] * (1.0 / N_NODES) - mean * mean
        scale = g_ref[...] * lax.rsqrt(var + 1e-5)
        shift = be_ref[...] - mean * scale
        o_ref[...] = o_ref[...] * scale + shift


def _tail(h0, parts, W1, b1, W2, b2, W3, b3, gamma, beta):
    W1p = jnp.pad(W1, ((0, D_PAD - D_HID), (0, 0)))
    blk = lambda i: (jnp.minimum(i, NTB - 1), 0)
    blk3 = lambda i: (0, jnp.minimum(i, NTB - 1), 0)
    full = lambda i: (0, 0)
    return pl.pallas_call(
        _tail_body,
        grid=(NTB + 1,),
        in_specs=[
            pl.BlockSpec((TB, D_HID), blk),
            pl.BlockSpec((NC, TB, D_PAD), blk3),
            pl.BlockSpec((D_PAD, D_HID), full),
            pl.BlockSpec((1, D_HID), full),
            pl.BlockSpec((D_HID, D_HID), full),
            pl.BlockSpec((1, D_HID), full),
            pl.BlockSpec((D_HID, D_TGT), full),
            pl.BlockSpec((1, D_TGT), full),
            pl.BlockSpec((1, D_TGT), full),
            pl.BlockSpec((1, D_TGT), full),
        ],
        out_specs=pl.BlockSpec((N_NODES, D_TGT), full),
        out_shape=jax.ShapeDtypeStruct((N_NODES, D_TGT), jnp.float32),
        scratch_shapes=[
            pltpu.VMEM((1, D_TGT), jnp.float32),
            pltpu.VMEM((1, D_TGT), jnp.float32),
        ],
    )(h0, parts, W1p, b1.reshape(1, D_HID), W2, b2.reshape(1, D_HID),
      W3, b3.reshape(1, D_TGT), gamma.reshape(1, D_TGT), beta.reshape(1, D_TGT))


def kernel(feature, edge_index, W0, b0, W1, b1, W2, b2, W3, b3, gamma, beta):
    ei = edge_index.astype(jnp.int32)
    h0 = _head(feature, W0, b0)
    parts = _seg_sum(h0, ei)
    return _tail(h0, parts, W1, b1, W2, b2, W3, b3, gamma, beta)
